# Initial kernel scaffold; baseline (speedup 1.0000x reference)
import jax
import jax.numpy as jnp
from jax.experimental import pallas as pl


def kernel(token_ids, weight):
    raise NotImplementedError("write your pallas kernel here")


# speedup vs baseline: 1.4028x; 1.4028x over previous
import functools

import jax
import jax.numpy as jnp
from jax import lax
from jax.experimental import pallas as pl
from jax.experimental.pallas import tpu as pltpu
from jax.experimental.pallas import tpu_sc as plsc


def _make_gather(num_rows: int, dim: int, nw: int, chunks: int, chunk_len: int):
    rows_per_w = chunks * chunk_len
    mesh = plsc.VectorSubcoreMesh(core_axis_name="c", subcore_axis_name="s")

    @functools.partial(
        pl.kernel,
        mesh=mesh,
        out_type=jax.ShapeDtypeStruct((nw * rows_per_w, dim), jnp.float32),
        compiler_params=pltpu.CompilerParams(use_tc_tiling_on_sc=False),
        scratch_types=[
            pltpu.VMEM((chunks, chunk_len), jnp.int32),
            pltpu.VMEM((chunk_len, dim), jnp.float32),
            pltpu.SemaphoreType.DMA,
        ],
    )
    def gather_kernel(idx_hbm, table_hbm, out_hbm, idx_v, rows_v, sem):
        wid = lax.axis_index("s") * 2 + lax.axis_index("c")
        base = wid * rows_per_w
        pltpu.sync_copy(idx_hbm.at[wid], idx_v)

        def body(j, carry):
            pltpu.async_copy(table_hbm.at[idx_v.at[j]], rows_v, sem).wait()
            pltpu.sync_copy(rows_v, out_hbm.at[pl.ds(base + j * chunk_len, chunk_len)])
            return carry

        lax.fori_loop(0, chunks, body, 0)

    return gather_kernel


def kernel(token_ids, weight):
    b, s = token_ids.shape
    num_rows, dim = weight.shape
    n = b * s
    nw = 32
    chunk_len = 128
    assert n % (nw * chunk_len) == 0
    chunks = n // (nw * chunk_len)
    ids = token_ids.reshape(nw, chunks, chunk_len).astype(jnp.int32)
    out = _make_gather(num_rows, dim, nw, chunks, chunk_len)(ids, weight)
    return out.reshape(b, s, dim)


# speedup vs baseline: 1.5131x; 1.0786x over previous
import functools

import jax
import jax.numpy as jnp
from jax import lax
from jax.experimental import pallas as pl
from jax.experimental.pallas import tpu as pltpu
from jax.experimental.pallas import tpu_sc as plsc

_CHUNK = 128
_SG = 8
_NBUF = 3


def _make_gather(dim: int, nw: int, chunks: int):
    rows_per_w = chunks * _CHUNK
    ngroups = chunks // _SG
    rows_per_g = _SG * _CHUNK
    mesh = plsc.VectorSubcoreMesh(core_axis_name="c", subcore_axis_name="s")

    @functools.partial(
        pl.kernel,
        mesh=mesh,
        out_type=jax.ShapeDtypeStruct((nw * rows_per_w, dim), jnp.float32),
        compiler_params=pltpu.CompilerParams(use_tc_tiling_on_sc=False),
        scratch_types=[
            pltpu.VMEM((chunks, _CHUNK), jnp.int32),
            *[pltpu.VMEM((rows_per_g, dim), jnp.float32) for _ in range(_NBUF)],
            *[pltpu.SemaphoreType.DMA for _ in range(2 * _NBUF)],
        ],
    )
    def gather_kernel(idx_hbm, table_hbm, out_hbm, idx_v, *scr):
        stage = scr[:_NBUF]
        gsem = scr[_NBUF:2 * _NBUF]
        wsem = scr[2 * _NBUF:]
        wid = lax.axis_index("s") * 2 + lax.axis_index("c")
        base = wid * rows_per_w
        pltpu.sync_copy(idx_hbm.at[wid], idx_v)

        def fire(g, p):
            return [
                pltpu.async_copy(
                    table_hbm.at[idx_v.at[g * _SG + i]],
                    stage[p].at[pl.ds(i * _CHUNK, _CHUNK)],
                    gsem[p],
                )
                for i in range(_SG)
            ]

        pending = [None] * _NBUF
        writes = [None] * _NBUF
        pending[0] = fire(0, 0)
        for g in range(ngroups):
            p = g % _NBUF
            if g + 1 < ngroups:
                q = (g + 1) % _NBUF
                if writes[q] is not None:
                    writes[q].wait()
                    writes[q] = None
                pending[q] = fire(g + 1, q)
            for c in pending[p]:
                c.wait()
            writes[p] = pltpu.async_copy(
                stage[p],
                out_hbm.at[pl.ds(base + g * rows_per_g, rows_per_g)],
                wsem[p],
            )
        for w in writes:
            if w is not None:
                w.wait()

    return gather_kernel


def kernel(token_ids, weight):
    b, s = token_ids.shape
    _, dim = weight.shape
    n = b * s
    nw = 32
    assert n % (nw * _CHUNK * _SG) == 0
    chunks = n // (nw * _CHUNK)
    ids = token_ids.reshape(nw, chunks, _CHUNK).astype(jnp.int32)
    out = _make_gather(dim, nw, chunks)(ids, weight)
    return out.reshape(b, s, dim)
